# baseline (speedup 1.0000x reference)
import functools

import jax
import jax.numpy as jnp
from jax import lax
from jax.experimental import pallas as pl
from jax.experimental.pallas import tpu as pltpu
from jax.experimental.pallas import tpu_sc as plsc

NC = 2
NS = 16
NW = NC * NS
CHUNK = 64


def _sc_gather(table, idx_flat, n_rows, hidden):
    per_w = n_rows // NW
    mesh = plsc.VectorSubcoreMesh(core_axis_name="c", subcore_axis_name="s")

    @functools.partial(
        pl.kernel,
        mesh=mesh,
        out_type=jax.ShapeDtypeStruct((n_rows, hidden), jnp.float32),
        scratch_types=[
            pltpu.VMEM((CHUNK,), jnp.int32),
            pltpu.VMEM((CHUNK,), jnp.int32),
            pltpu.VMEM((CHUNK, hidden), jnp.float32),
            pltpu.VMEM((CHUNK, hidden), jnp.float32),
            pltpu.SemaphoreType.DMA,
            pltpu.SemaphoreType.DMA,
            pltpu.SemaphoreType.DMA,
            pltpu.SemaphoreType.DMA,
        ],
    )
    def gather_kernel(table_hbm, idx_hbm, out_hbm, idx0, idx1, rows0, rows1,
                      sg0, sg1, sw0, sw1):
        wid = lax.axis_index("s") * NC + lax.axis_index("c")
        base = wid * per_w

        @pl.loop(0, per_w, step=2 * CHUNK)
        def _(off):
            @pl.when(off > 0)
            def _():
                pltpu.make_async_copy(
                    rows0, out_hbm.at[pl.ds(base + off - 2 * CHUNK, CHUNK)],
                    sw0).wait()
                pltpu.make_async_copy(
                    rows1, out_hbm.at[pl.ds(base + off - CHUNK, CHUNK)],
                    sw1).wait()

            pltpu.sync_copy(idx_hbm.at[pl.ds(base + off, CHUNK)], idx0)
            g0 = pltpu.async_copy(table_hbm.at[idx0], rows0, sg0)
            pltpu.sync_copy(idx_hbm.at[pl.ds(base + off + CHUNK, CHUNK)], idx1)
            g1 = pltpu.async_copy(table_hbm.at[idx1], rows1, sg1)
            g0.wait()
            pltpu.async_copy(rows0, out_hbm.at[pl.ds(base + off, CHUNK)], sw0)
            g1.wait()
            pltpu.async_copy(rows1, out_hbm.at[pl.ds(base + off + CHUNK, CHUNK)],
                             sw1)

        pltpu.make_async_copy(
            rows0, out_hbm.at[pl.ds(base + per_w - 2 * CHUNK, CHUNK)], sw0).wait()
        pltpu.make_async_copy(
            rows1, out_hbm.at[pl.ds(base + per_w - CHUNK, CHUNK)], sw1).wait()

    return gather_kernel(table, idx_flat)


def _tc_body(g_ref, tt_ref, pos_ref, type_ref, gamma_ref, beta_ref, o_ref):
    x = g_ref[...]
    tt = tt_ref[...][..., None]
    t0 = type_ref[0]
    t1 = type_ref[1]
    t2 = type_ref[2]
    type_emb = jnp.where(tt == 0, t0, jnp.where(tt == 1, t1, t2))
    x = x + pos_ref[...][None] + type_emb
    mean = jnp.mean(x, axis=-1, keepdims=True)
    xc = x - mean
    var = jnp.mean(xc * xc, axis=-1, keepdims=True)
    inv = lax.rsqrt(var + 1e-12)
    o_ref[...] = xc * inv * gamma_ref[...] + beta_ref[...]


def _tc_body_acc(g_ref, tt_ref, pos_ref, type_ref, gamma_ref, beta_ref, prev_ref,
                 o_ref):
    del prev_ref
    _tc_body(g_ref, tt_ref, pos_ref, type_ref, gamma_ref, beta_ref, o_ref)


def _tc_add_ln(gathered, token_type_ids, pos_table, type_pad, gamma, beta,
               full_batch, chunk, prev):
    bc, S = token_type_ids.shape
    H = gathered.shape[-1]
    SB = 8
    nblk = bc // SB
    base = chunk // SB
    in_specs = [
        pl.BlockSpec((SB, S, H), lambda i: (i, 0, 0)),
        pl.BlockSpec((SB, S), lambda i: (i, 0)),
        pl.BlockSpec((S, H), lambda i: (0, 0)),
        pl.BlockSpec((8, H), lambda i: (0, 0)),
        pl.BlockSpec((H,), lambda i: (0,)),
        pl.BlockSpec((H,), lambda i: (0,)),
    ]
    args = [gathered, token_type_ids, pos_table, type_pad, gamma, beta]
    body = _tc_body
    alias = {}
    if prev is not None:
        in_specs.append(pl.BlockSpec(memory_space=pl.ANY))
        args.append(prev)
        body = _tc_body_acc
        alias = {6: 0}
    return pl.pallas_call(
        body,
        grid=(nblk,),
        in_specs=in_specs,
        out_specs=pl.BlockSpec((SB, S, H), lambda i: (base + i, 0, 0)),
        out_shape=jax.ShapeDtypeStruct((full_batch, S, H), jnp.float32),
        input_output_aliases=alias,
        compiler_params=pltpu.CompilerParams(
            dimension_semantics=("parallel",),
        ),
    )(*args)


CHUNK_SIZES = (128,) * 8


def kernel(input_ids, token_type_ids, tok_table, pos_table, type_table, gamma, beta):
    B, S = input_ids.shape
    V, H = tok_table.shape
    type_pad = jnp.zeros((8, H), jnp.float32).at[:3].set(type_table)

    gathered = []
    starts = []
    s0 = 0
    for bc in CHUNK_SIZES:
        ids_c = input_ids[s0:s0 + bc].reshape(bc * S).astype(jnp.int32)
        gathered.append(_sc_gather(tok_table, ids_c, bc * S, H).reshape(bc, S, H))
        starts.append(s0)
        s0 += bc
    out = None
    for g_c, s0, bc in zip(gathered, starts, CHUNK_SIZES):
        tt_c = token_type_ids[s0:s0 + bc]
        out = _tc_add_ln(g_c, tt_c, pos_table, type_pad, gamma, beta,
                         B, s0, out)
    return out


# speedup vs baseline: 1.0281x; 1.0281x over previous
import functools

import jax
import jax.numpy as jnp
from jax import lax
from jax.experimental import pallas as pl
from jax.experimental.pallas import tpu as pltpu
from jax.experimental.pallas import tpu_sc as plsc

NC = 2
NS = 16
NW = NC * NS
CHUNK = 64


def _sc_gather(table, idx_flat, n_rows, hidden):
    per_w = n_rows // NW
    mesh = plsc.VectorSubcoreMesh(core_axis_name="c", subcore_axis_name="s")

    @functools.partial(
        pl.kernel,
        mesh=mesh,
        out_type=jax.ShapeDtypeStruct((n_rows, hidden), jnp.float32),
        scratch_types=[
            pltpu.VMEM((CHUNK,), jnp.int32),
            pltpu.VMEM((CHUNK,), jnp.int32),
            pltpu.VMEM((CHUNK, hidden), jnp.float32),
            pltpu.VMEM((CHUNK, hidden), jnp.float32),
            pltpu.SemaphoreType.DMA,
            pltpu.SemaphoreType.DMA,
            pltpu.SemaphoreType.DMA,
            pltpu.SemaphoreType.DMA,
        ],
    )
    def gather_kernel(table_hbm, idx_hbm, out_hbm, idx0, idx1, rows0, rows1,
                      sg0, sg1, sw0, sw1):
        wid = lax.axis_index("s") * NC + lax.axis_index("c")
        base = wid * per_w

        @pl.loop(0, per_w, step=2 * CHUNK)
        def _(off):
            @pl.when(off > 0)
            def _():
                pltpu.make_async_copy(
                    rows0, out_hbm.at[pl.ds(base + off - 2 * CHUNK, CHUNK)],
                    sw0).wait()
                pltpu.make_async_copy(
                    rows1, out_hbm.at[pl.ds(base + off - CHUNK, CHUNK)],
                    sw1).wait()

            pltpu.sync_copy(idx_hbm.at[pl.ds(base + off, CHUNK)], idx0)
            g0 = pltpu.async_copy(table_hbm.at[idx0], rows0, sg0)
            pltpu.sync_copy(idx_hbm.at[pl.ds(base + off + CHUNK, CHUNK)], idx1)
            g1 = pltpu.async_copy(table_hbm.at[idx1], rows1, sg1)
            g0.wait()
            pltpu.async_copy(rows0, out_hbm.at[pl.ds(base + off, CHUNK)], sw0)
            g1.wait()
            pltpu.async_copy(rows1, out_hbm.at[pl.ds(base + off + CHUNK, CHUNK)],
                             sw1)

        pltpu.make_async_copy(
            rows0, out_hbm.at[pl.ds(base + per_w - 2 * CHUNK, CHUNK)], sw0).wait()
        pltpu.make_async_copy(
            rows1, out_hbm.at[pl.ds(base + per_w - CHUNK, CHUNK)], sw1).wait()

    return gather_kernel(table, idx_flat)


def _tc_body(g_ref, tt_ref, pos_ref, type_ref, gamma_ref, beta_ref, o_ref):
    x = g_ref[...]
    tt = tt_ref[...][..., None]
    t0 = type_ref[0]
    t1 = type_ref[1]
    t2 = type_ref[2]
    type_emb = jnp.where(tt == 0, t0, jnp.where(tt == 1, t1, t2))
    x = x + pos_ref[...][None] + type_emb
    mean = jnp.mean(x, axis=-1, keepdims=True)
    xc = x - mean
    var = jnp.mean(xc * xc, axis=-1, keepdims=True)
    inv = lax.rsqrt(var + 1e-12)
    o_ref[...] = xc * inv * gamma_ref[...] + beta_ref[...]


def _tc_body_acc(g_ref, tt_ref, pos_ref, type_ref, gamma_ref, beta_ref, prev_ref,
                 o_ref):
    del prev_ref
    _tc_body(g_ref, tt_ref, pos_ref, type_ref, gamma_ref, beta_ref, o_ref)


def _tc_add_ln(gathered, token_type_ids, pos_table, type_pad, gamma, beta,
               full_batch, chunk, prev):
    bc, S = token_type_ids.shape
    H = gathered.shape[-1]
    SB = 8
    nblk = bc // SB
    base = chunk // SB
    in_specs = [
        pl.BlockSpec((SB, S, H), lambda i: (i, 0, 0)),
        pl.BlockSpec((SB, S), lambda i: (i, 0)),
        pl.BlockSpec((S, H), lambda i: (0, 0)),
        pl.BlockSpec((8, H), lambda i: (0, 0)),
        pl.BlockSpec((H,), lambda i: (0,)),
        pl.BlockSpec((H,), lambda i: (0,)),
    ]
    args = [gathered, token_type_ids, pos_table, type_pad, gamma, beta]
    body = _tc_body
    alias = {}
    if prev is not None:
        in_specs.append(pl.BlockSpec(memory_space=pl.ANY))
        args.append(prev)
        body = _tc_body_acc
        alias = {6: 0}
    return pl.pallas_call(
        body,
        grid=(nblk,),
        in_specs=in_specs,
        out_specs=pl.BlockSpec((SB, S, H), lambda i: (base + i, 0, 0)),
        out_shape=jax.ShapeDtypeStruct((full_batch, S, H), jnp.float32),
        input_output_aliases=alias,
        compiler_params=pltpu.CompilerParams(
            dimension_semantics=("parallel",),
        ),
    )(*args)


CHUNK_SIZES = (512, 512)


def kernel(input_ids, token_type_ids, tok_table, pos_table, type_table, gamma, beta):
    B, S = input_ids.shape
    V, H = tok_table.shape
    type_pad = jnp.zeros((8, H), jnp.float32).at[:3].set(type_table)

    gathered = []
    starts = []
    s0 = 0
    for bc in CHUNK_SIZES:
        ids_c = input_ids[s0:s0 + bc].reshape(bc * S).astype(jnp.int32)
        gathered.append(_sc_gather(tok_table, ids_c, bc * S, H).reshape(bc, S, H))
        starts.append(s0)
        s0 += bc
    out = None
    for g_c, s0, bc in zip(gathered, starts, CHUNK_SIZES):
        tt_c = token_type_ids[s0:s0 + bc]
        out = _tc_add_ln(g_c, tt_c, pos_table, type_pad, gamma, beta,
                         B, s0, out)
    return out
